# baseline (speedup 1.0000x reference)
import functools

import jax
import jax.numpy as jnp
from jax import lax
from jax.experimental import pallas as pl
from jax.experimental.pallas import tpu as pltpu
from jax.experimental.pallas import tpu_sc as plsc

LANES = 16
CHUNK = 16
NBUF = 4
EPS = 1e-12

_DNUMS = lax.GatherDimensionNumbers(
    offset_dims=(), collapsed_slice_dims=(0,), start_index_map=(0,))


def _shuffle(x, perm):
    return lax.gather(x, perm.reshape(LANES, 1), _DNUMS, slice_sizes=(1,),
                      mode=lax.GatherScatterMode.PROMISE_IN_BOUNDS)


def _lane_sum(x):
    iota = lax.iota(jnp.int32, LANES)
    for k in (8, 4, 2, 1):
        x = x + _shuffle(x, lax.bitwise_xor(iota, k))
    return x


def _lane_bcast(x, r):
    return _shuffle(x, lax.broadcast_in_dim(r, (LANES,), ()))


def _rsqrt(v):
    bits = lax.bitcast_convert_type(v, jnp.int32)
    ones = jnp.full((LANES,), 1, jnp.int32)
    bits = 0x5F3759DF - lax.shift_right_logical(bits, ones)
    y = lax.bitcast_convert_type(bits, jnp.float32)
    half = v * 0.5
    for _ in range(2):
        y = y * (1.5 - half * y * y)
    return y


@functools.lru_cache(maxsize=None)
def _make_sc_kernel(n_tokens, seq, hidden):
    info = plsc.get_sparse_core_info()
    n_workers = info.num_cores * info.num_subcores
    assert n_tokens % (n_workers * seq) == 0, "each worker owns whole sequences"
    tok_per_w = n_tokens // n_workers
    seq_per_w = tok_per_w // seq
    n_chunks = tok_per_w // CHUNK
    assert n_chunks % NBUF == 0
    assert seq % CHUNK == 0 and hidden % (4 * LANES) == 0
    hchunks = hidden // LANES
    inv_h = 1.0 / hidden

    @functools.partial(
        pl.kernel,
        out_type=jax.ShapeDtypeStruct((n_tokens, hidden), jnp.float32),
        mesh=plsc.VectorSubcoreMesh(core_axis_name="c", subcore_axis_name="s"),
        scratch_types=[
            pltpu.VMEM((tok_per_w,), jnp.int32),
            pltpu.VMEM((n_chunks,), jnp.int32),
            pltpu.VMEM((CHUNK, hidden), jnp.float32),
            pltpu.VMEM((NBUF * CHUNK, hidden), jnp.float32),
            pltpu.VMEM((hidden,), jnp.float32),
            pltpu.VMEM((hidden,), jnp.float32),
            pltpu.VMEM((hidden,), jnp.float32),
            pltpu.VMEM((hidden,), jnp.float32),
            pltpu.SemaphoreType.DMA,
            pltpu.SemaphoreType.DMA,
            pltpu.SemaphoreType.DMA,
            pltpu.SemaphoreType.DMA,
        ],
    )
    def sc_kernel(ids_hbm, tpk_hbm, word_hbm, pose0_hbm, dt_hbm, gamma_hbm,
                  beta_hbm, out_hbm, ids_v, tpk_v, pos_v, rows_v,
                  gamma_v, beta_v, dt_v, xbuf, gsem0, gsem1, wsem0, wsem1):
        wid = lax.axis_index("s") * info.num_cores + lax.axis_index("c")
        tok0 = wid * tok_per_w
        pltpu.sync_copy(ids_hbm.at[pl.ds(tok0, tok_per_w)], ids_v)
        pltpu.sync_copy(tpk_hbm.at[pl.ds(wid * n_chunks, n_chunks)], tpk_v)
        pltpu.sync_copy(gamma_hbm, gamma_v)
        pltpu.sync_copy(beta_hbm, beta_v)
        pltpu.sync_copy(dt_hbm, dt_v)
        iota = lax.iota(jnp.int32, LANES)

        def chunk_off(t):
            b = lax.rem(t, seq_per_w)
            j = t // seq_per_w
            return b, j, b * seq + j * CHUNK

        def issue_gather(t, gsem):
            _, _, off = chunk_off(t)
            base = lax.rem(t, NBUF) * CHUNK
            pltpu.async_copy(word_hbm.at[ids_v.at[pl.ds(off, CHUNK)]],
                             rows_v.at[pl.ds(base, CHUNK)], gsem)
            return None

        def wait_gather(gsem):
            pltpu.make_async_copy(word_hbm.at[pl.ds(0, CHUNK)],
                                  rows_v.at[pl.ds(0, CHUNK)], gsem).wait()

        def wait_write(wsem):
            pltpu.make_async_copy(rows_v.at[pl.ds(0, CHUNK)],
                                  out_hbm.at[pl.ds(0, CHUNK)], wsem).wait()

        issue_gather(jnp.int32(0), gsem0)
        issue_gather(jnp.int32(1), gsem1)

        def chunk_body(t, carry):
            b, j, off = chunk_off(t)
            base = lax.rem(t, NBUF) * CHUNK
            even = lax.rem(t, 2) == 0
            pl.when(b == 0)(
                lambda: pltpu.sync_copy(pose0_hbm.at[pl.ds(j * CHUNK, CHUNK)],
                                        pos_v))

            def dma_front(gsem, wsem):
                def run():
                    wait_gather(gsem)
                    pl.when(t >= 2)(lambda: wait_write(wsem))
                    pl.when(t + 2 < n_chunks)(lambda: issue_gather(t + 2, gsem))
                return run

            pl.when(even)(dma_front(gsem0, wsem0))
            pl.when(jnp.logical_not(even))(dma_front(gsem1, wsem1))

            twords = tpk_v[pl.ds((t // LANES) * LANES, LANES)]
            tword = _lane_bcast(twords, lax.rem(t, LANES))
            tf = (lax.shift_right_logical(tword, iota) & 1).astype(jnp.float32)

            def row_body(r, rcarry):
                row = base + r
                t_r = _lane_bcast(tf, r)
                zero = jnp.zeros((LANES,), jnp.float32)

                sl0 = pl.ds(0, LANES)
                prefetch0 = (rows_v[row, sl0], pos_v[r, sl0], dt_v[sl0])

                @plsc.parallel_loop(0, hchunks, carry=prefetch0 + (zero, zero),
                                    unroll=4)
                def p1(ci, carry):
                    wv, pv, dv, a, q = carry
                    nxt = jnp.minimum(ci + 1, hchunks - 1) * LANES
                    sln = pl.ds(nxt, LANES)
                    x = wv + pv + t_r * dv
                    xbuf[pl.ds(ci * LANES, LANES)] = x
                    return (rows_v[row, sln], pos_v[r, sln], dt_v[sln],
                            a + x, q + x * x)

                accv = p1[3]
                accsqv = p1[4]
                mean_v = _lane_sum(accv) * inv_h
                var_v = _lane_sum(accsqv) * inv_h - mean_v * mean_v
                y = _rsqrt(var_v + EPS)

                pre2 = (xbuf[sl0], gamma_v[sl0], beta_v[sl0])

                @plsc.parallel_loop(0, hchunks, carry=pre2, unroll=4)
                def p2(c, carry):
                    xv, gv, bv, = carry
                    nxt = jnp.minimum(c + 1, hchunks - 1) * LANES
                    sln = pl.ds(nxt, LANES)
                    xhat = (xv - mean_v) * y
                    rows_v[row, pl.ds(c * LANES, LANES)] = xhat * gv + bv
                    return (xbuf[sln], gamma_v[sln], beta_v[sln])

                return rcarry

            lax.fori_loop(0, CHUNK, row_body, 0)
            wout = out_hbm.at[pl.ds(tok0 + off, CHUNK)]
            src = rows_v.at[pl.ds(base, CHUNK)]

            def start_write(wsem):
                def run():
                    pltpu.async_copy(src, wout, wsem)
                return run

            pl.when(even)(start_write(wsem0))
            pl.when(jnp.logical_not(even))(start_write(wsem1))
            return carry

        lax.fori_loop(0, n_chunks, chunk_body, 0)
        wait_write(wsem0)
        wait_write(wsem1)

    return sc_kernel


def kernel(input_ids, token_type_ids, word_embeddings, position_embeddings,
           token_type_embeddings, gamma, beta):
    batch, seq = input_ids.shape
    hidden = word_embeddings.shape[1]
    n_tokens = batch * seq
    ids = input_ids.reshape(-1).astype(jnp.int32)
    tids = token_type_ids.reshape(-1).astype(jnp.int32)
    info = plsc.get_sparse_core_info()
    n_workers = info.num_cores * info.num_subcores
    seq_per_w = n_tokens // seq // n_workers
    slots = seq // CHUNK
    tpk = (tids.reshape(n_workers, seq_per_w, slots, LANES)
           * (1 << jnp.arange(LANES, dtype=jnp.int32))).sum(
               axis=-1, dtype=jnp.int32).transpose(0, 2, 1).reshape(-1)
    pose0 = position_embeddings[:seq] + token_type_embeddings[0][None, :]
    dt = token_type_embeddings[1] - token_type_embeddings[0]
    sc = _make_sc_kernel(n_tokens, seq, hidden)
    out = sc(ids, tpk, word_embeddings, pose0, dt, gamma, beta)
    return out.reshape(batch, seq, hidden)


# speedup vs baseline: 1.5706x; 1.5706x over previous
import functools

import jax
import jax.numpy as jnp
from jax import lax
from jax.experimental import pallas as pl
from jax.experimental.pallas import tpu as pltpu
from jax.experimental.pallas import tpu_sc as plsc

LANES = 16
CHUNK = 16
NBUF = 4
EPS = 1e-12

_DNUMS = lax.GatherDimensionNumbers(
    offset_dims=(), collapsed_slice_dims=(0,), start_index_map=(0,))


def _shuffle(x, perm):
    return lax.gather(x, perm.reshape(LANES, 1), _DNUMS, slice_sizes=(1,),
                      mode=lax.GatherScatterMode.PROMISE_IN_BOUNDS)


def _lane_sum(x):
    iota = lax.iota(jnp.int32, LANES)
    for k in (8, 4, 2, 1):
        x = x + _shuffle(x, lax.bitwise_xor(iota, k))
    return x


def _lane_bcast(x, r):
    return _shuffle(x, lax.broadcast_in_dim(r, (LANES,), ()))


def _rsqrt(v):
    bits = lax.bitcast_convert_type(v, jnp.int32)
    ones = jnp.full((LANES,), 1, jnp.int32)
    bits = 0x5F3759DF - lax.shift_right_logical(bits, ones)
    y = lax.bitcast_convert_type(bits, jnp.float32)
    half = v * 0.5
    for _ in range(2):
        y = y * (1.5 - half * y * y)
    return y


@functools.lru_cache(maxsize=None)
def _make_sc_kernel(n_tokens, seq, hidden):
    info = plsc.get_sparse_core_info()
    n_workers = info.num_cores * info.num_subcores
    assert n_tokens % (n_workers * seq) == 0, "each worker owns whole sequences"
    tok_per_w = n_tokens // n_workers
    seq_per_w = tok_per_w // seq
    n_chunks = tok_per_w // CHUNK
    assert n_chunks % NBUF == 0
    assert seq % CHUNK == 0 and hidden % (4 * LANES) == 0
    hchunks = hidden // LANES
    inv_h = 1.0 / hidden

    @functools.partial(
        pl.kernel,
        out_type=jax.ShapeDtypeStruct((n_tokens, hidden), jnp.float32),
        mesh=plsc.VectorSubcoreMesh(core_axis_name="c", subcore_axis_name="s"),
        scratch_types=[
            pltpu.VMEM((tok_per_w,), jnp.int32),
            pltpu.VMEM((n_chunks,), jnp.int32),
            pltpu.VMEM((CHUNK, hidden), jnp.float32),
            pltpu.VMEM((NBUF * CHUNK, hidden), jnp.float32),
            pltpu.VMEM((hidden,), jnp.float32),
            pltpu.VMEM((hidden,), jnp.float32),
            pltpu.VMEM((hidden,), jnp.float32),
            pltpu.SemaphoreType.DMA,
            pltpu.SemaphoreType.DMA,
            pltpu.SemaphoreType.DMA,
            pltpu.SemaphoreType.DMA,
        ],
    )
    def sc_kernel(ids_hbm, tpk_hbm, word_hbm, pose0_hbm, dt_hbm, gamma_hbm,
                  beta_hbm, out_hbm, ids_v, tpk_v, pos_v, rows_v,
                  gamma_v, beta_v, dt_v, gsem0, gsem1, wsem0, wsem1):
        wid = lax.axis_index("s") * info.num_cores + lax.axis_index("c")
        tok0 = wid * tok_per_w
        pltpu.sync_copy(ids_hbm.at[pl.ds(tok0, tok_per_w)], ids_v)
        pltpu.sync_copy(tpk_hbm.at[pl.ds(wid * n_chunks, n_chunks)], tpk_v)
        pltpu.sync_copy(gamma_hbm, gamma_v)
        pltpu.sync_copy(beta_hbm, beta_v)
        pltpu.sync_copy(dt_hbm, dt_v)
        iota = lax.iota(jnp.int32, LANES)

        def chunk_off(t):
            b = lax.rem(t, seq_per_w)
            j = t // seq_per_w
            return b, j, b * seq + j * CHUNK

        def issue_gather(t, gsem):
            _, _, off = chunk_off(t)
            base = lax.rem(t, NBUF) * CHUNK
            pltpu.async_copy(word_hbm.at[ids_v.at[pl.ds(off, CHUNK)]],
                             rows_v.at[pl.ds(base, CHUNK)], gsem)
            return None

        def wait_gather(gsem):
            pltpu.make_async_copy(word_hbm.at[pl.ds(0, CHUNK)],
                                  rows_v.at[pl.ds(0, CHUNK)], gsem).wait()

        def wait_write(wsem):
            pltpu.make_async_copy(rows_v.at[pl.ds(0, CHUNK)],
                                  out_hbm.at[pl.ds(0, CHUNK)], wsem).wait()

        issue_gather(jnp.int32(0), gsem0)
        issue_gather(jnp.int32(1), gsem1)

        def chunk_body(t, carry):
            b, j, off = chunk_off(t)
            base = lax.rem(t, NBUF) * CHUNK
            even = lax.rem(t, 2) == 0
            pl.when(b == 0)(
                lambda: pltpu.sync_copy(pose0_hbm.at[pl.ds(j * CHUNK, CHUNK)],
                                        pos_v))

            def dma_front(gsem, wsem):
                def run():
                    wait_gather(gsem)
                    pl.when(t >= 2)(lambda: wait_write(wsem))
                    pl.when(t + 2 < n_chunks)(lambda: issue_gather(t + 2, gsem))
                return run

            pl.when(even)(dma_front(gsem0, wsem0))
            pl.when(jnp.logical_not(even))(dma_front(gsem1, wsem1))

            twords = tpk_v[pl.ds((t // LANES) * LANES, LANES)]
            tword = _lane_bcast(twords, lax.rem(t, LANES))
            tf = (lax.shift_right_logical(tword, iota) & 1).astype(jnp.float32)

            def row_body(r, rcarry):
                row = base + r
                t_r = _lane_bcast(tf, r)
                zero = jnp.zeros((LANES,), jnp.float32)

                @plsc.parallel_loop(0, hchunks // 4, carry=(zero,) * 8,
                                    unroll=4)
                def p1(ci, acc):
                    accs = list(acc)
                    for k in range(4):
                        sl = pl.ds((ci * 4 + k) * LANES, LANES)
                        x = rows_v[row, sl] + pos_v[r, sl] + t_r * dt_v[sl]
                        rows_v[row, sl] = x
                        accs[k] = accs[k] + x
                        accs[4 + k] = accs[4 + k] + x * x
                    return tuple(accs)

                accv = (p1[0] + p1[1]) + (p1[2] + p1[3])
                accsqv = (p1[4] + p1[5]) + (p1[6] + p1[7])
                mean_v = _lane_sum(accv) * inv_h
                var_v = _lane_sum(accsqv) * inv_h - mean_v * mean_v
                y = _rsqrt(var_v + EPS)

                @plsc.parallel_loop(0, hchunks, unroll=8)
                def p2(c):
                    sl = pl.ds(c * LANES, LANES)
                    xhat = (rows_v[row, sl] - mean_v) * y
                    rows_v[row, sl] = xhat * gamma_v[sl] + beta_v[sl]

                return rcarry

            lax.fori_loop(0, CHUNK, row_body, 0)
            wout = out_hbm.at[pl.ds(tok0 + off, CHUNK)]
            src = rows_v.at[pl.ds(base, CHUNK)]

            def start_write(wsem):
                def run():
                    pltpu.async_copy(src, wout, wsem)
                return run

            pl.when(even)(start_write(wsem0))
            pl.when(jnp.logical_not(even))(start_write(wsem1))
            return carry

        lax.fori_loop(0, n_chunks, chunk_body, 0)
        wait_write(wsem0)
        wait_write(wsem1)

    return sc_kernel


def kernel(input_ids, token_type_ids, word_embeddings, position_embeddings,
           token_type_embeddings, gamma, beta):
    batch, seq = input_ids.shape
    hidden = word_embeddings.shape[1]
    n_tokens = batch * seq
    ids = input_ids.reshape(-1).astype(jnp.int32)
    tids = token_type_ids.reshape(-1).astype(jnp.int32)
    info = plsc.get_sparse_core_info()
    n_workers = info.num_cores * info.num_subcores
    seq_per_w = n_tokens // seq // n_workers
    slots = seq // CHUNK
    tpk = (tids.reshape(n_workers, seq_per_w, slots, LANES)
           * (1 << jnp.arange(LANES, dtype=jnp.int32))).sum(
               axis=-1, dtype=jnp.int32).transpose(0, 2, 1).reshape(-1)
    pose0 = position_embeddings[:seq] + token_type_embeddings[0][None, :]
    dt = token_type_embeddings[1] - token_type_embeddings[0]
    sc = _make_sc_kernel(n_tokens, seq, hidden)
    out = sc(ids, tpk, word_embeddings, pose0, dt, gamma, beta)
    return out.reshape(batch, seq, hidden)
